# baseline (speedup 1.0000x reference)
import functools

import jax
import jax.numpy as jnp
from jax import lax
from jax.experimental import pallas as pl
from jax.experimental.pallas import tpu as pltpu
from jax.experimental.pallas import tpu_sc as plsc

B, S, D, H = 4, 8192, 1024, 1024
NW = 32
PER_TILE = (B * S) // NW
CH = 8
NCH = PER_TILE // CH
DCH = D // 16
NACC = 8
NBUF = 8


def _pool_body(ids_hbm, table_hbm, out_hbm, idx_v, rows0_v, rows1_v, rows2_v,
               rows3_v, rows4_v, rows5_v, rows6_v, rows7_v, acc_v,
               sem0, sem1, sem2, sem3, sem4, sem5, sem6, sem7):
  c = lax.axis_index("c")
  s = lax.axis_index("s")
  w = s * 2 + c
  b = w % B
  k = w // B
  base = b * S + k * PER_TILE

  bufs = (rows0_v, rows1_v, rows2_v, rows3_v, rows4_v, rows5_v, rows6_v,
          rows7_v)
  sems = (sem0, sem1, sem2, sem3, sem4, sem5, sem6, sem7)

  pltpu.sync_copy(ids_hbm.at[pl.ds(base, PER_TILE)], idx_v)

  def start(j, buf, sem):
    pltpu.async_copy(table_hbm.at[idx_v.at[pl.ds(j * CH, CH)]], buf, sem)

  def wait(j, buf, sem):
    pltpu.make_async_copy(
        table_hbm.at[idx_v.at[pl.ds(j * CH, CH)]], buf, sem).wait()

  @plsc.parallel_loop(0, DCH, 1, unroll=4)
  def zero_body(ci):
    acc_v[0, pl.ds(ci * 16, 16)] = jnp.zeros((16,), jnp.float32)

  def accumulate(buf):
    @plsc.parallel_loop(0, DCH, 1, unroll=2)
    def col_body(ci):
      sl = pl.ds(ci * 16, 16)
      vs = [buf[r, sl] for r in range(NACC)]
      for r in range(NACC, CH):
        vs[r % NACC] = vs[r % NACC] + buf[r, sl]
      t0 = (vs[0] + vs[1]) + (vs[2] + vs[3])
      t1 = (vs[4] + vs[5]) + (vs[6] + vs[7])
      plsc.addupdate(acc_v.at[0, sl], t0 + t1)

  for p in range(NBUF - 1):
    start(p, bufs[p], sems[p])

  def ring_body(t, carry):
    j = NBUF * t
    for q in range(NBUF):
      jq = j + q
      nxt = (q + NBUF - 1) % NBUF

      @pl.when(jq + NBUF - 1 < NCH)
      def _():
        start(jq + NBUF - 1, bufs[nxt], sems[nxt])

      wait(jq, bufs[q], sems[q])
      accumulate(bufs[q])
    return carry

  lax.fori_loop(0, NCH // NBUF, ring_body, 0)
  pltpu.sync_copy(acc_v, out_hbm.at[pl.ds(w, 1)])


_pool = functools.partial(
    pl.kernel,
    mesh=plsc.VectorSubcoreMesh(core_axis_name="c", subcore_axis_name="s"),
    out_type=jax.ShapeDtypeStruct((NW, D), jnp.float32),
    scratch_types=(
        [pltpu.VMEM((PER_TILE,), jnp.int32)]
        + [pltpu.VMEM((CH, D), jnp.float32) for _ in range(NBUF)]
        + [pltpu.VMEM((1, D), jnp.float32)]
        + [pltpu.SemaphoreType.DMA for _ in range(NBUF)]
    ),
)(_pool_body)


def _head_body(p_ref, mask_ref, row0_ref, w1_ref, b1_ref, w2_ref, b2_ref,
               out_ref):
  p = p_ref[...]
  r1 = p[0:16] + p[16:32]
  r2 = r1[0:8] + r1[8:16]
  feats = r2[0:4] + r2[4:8]

  msum = jnp.sum(mask_ref[...].astype(jnp.float32), axis=1, keepdims=True)
  feats = feats - (float(S) - msum) * row0_ref[...]
  feats = feats / jnp.clip(msum, 1e-6, None)

  h = jnp.dot(feats, w1_ref[...], preferred_element_type=jnp.float32)
  h = h + b1_ref[...]
  h = h * jax.nn.sigmoid(h)
  out_ref[...] = jnp.sum(h * w2_ref[...], axis=1, keepdims=True) + b2_ref[...]


_head = pl.pallas_call(
    _head_body,
    out_shape=jax.ShapeDtypeStruct((B, 1), jnp.float32),
)


def kernel(input_ids, attention_mask, table, W1, b1, W2, b2):
  mask = attention_mask.astype(jnp.int32)
  ids = jnp.where(mask != 0, input_ids.astype(jnp.int32), 0).reshape(-1)
  partials = _pool(ids, table)
  out = _head(
      partials,
      mask,
      table[0:1, :],
      W1,
      b1.reshape(1, H),
      W2.reshape(1, H),
      b2.reshape(1, 1),
  )
  return out.reshape(B)


# speedup vs baseline: 1.0152x; 1.0152x over previous
import functools

import jax
import jax.numpy as jnp
from jax import lax
from jax.experimental import pallas as pl
from jax.experimental.pallas import tpu as pltpu
from jax.experimental.pallas import tpu_sc as plsc

B, S, D, H = 4, 8192, 1024, 1024
NW = 32
PER_TILE = (B * S) // NW
CH = 16
NCH = PER_TILE // CH
DCH = D // 16
NACC = 8


def _pool_body(ids_hbm, table_hbm, out_hbm, idx_v, rows0_v, rows1_v, rows2_v,
               rows3_v, acc_v, sem0, sem1, sem2, sem3):
  c = lax.axis_index("c")
  s = lax.axis_index("s")
  w = s * 2 + c
  b = w % B
  k = w // B
  base = b * S + k * PER_TILE

  bufs = (rows0_v, rows1_v, rows2_v, rows3_v)
  sems = (sem0, sem1, sem2, sem3)

  pltpu.sync_copy(ids_hbm.at[pl.ds(base, PER_TILE)], idx_v)

  def start(j, buf, sem):
    pltpu.async_copy(table_hbm.at[idx_v.at[pl.ds(j * CH, CH)]], buf, sem)

  def wait(j, buf, sem):
    pltpu.make_async_copy(
        table_hbm.at[idx_v.at[pl.ds(j * CH, CH)]], buf, sem).wait()

  @plsc.parallel_loop(0, DCH, 1, unroll=4)
  def zero_body(ci):
    acc_v[0, pl.ds(ci * 16, 16)] = jnp.zeros((16,), jnp.float32)

  def accumulate(buf):
    @plsc.parallel_loop(0, DCH, 1, unroll=2)
    def col_body(ci):
      sl = pl.ds(ci * 16, 16)
      vs = [buf[r, sl] for r in range(NACC)]
      for r in range(NACC, CH):
        vs[r % NACC] = vs[r % NACC] + buf[r, sl]
      t0 = (vs[0] + vs[1]) + (vs[2] + vs[3])
      t1 = (vs[4] + vs[5]) + (vs[6] + vs[7])
      plsc.addupdate(acc_v.at[0, sl], t0 + t1)

  for p in range(3):
    start(p, bufs[p], sems[p])

  def quad_body(t, carry):
    j = 4 * t
    for q in range(4):
      jq = j + q
      nxt = (q + 3) % 4

      @pl.when(jq + 3 < NCH)
      def _():
        start(jq + 3, bufs[nxt], sems[nxt])

      wait(jq, bufs[q], sems[q])
      accumulate(bufs[q])
    return carry

  lax.fori_loop(0, NCH // 4, quad_body, 0)
  pltpu.sync_copy(acc_v, out_hbm.at[pl.ds(w, 1)])


_pool = functools.partial(
    pl.kernel,
    mesh=plsc.VectorSubcoreMesh(core_axis_name="c", subcore_axis_name="s"),
    out_type=jax.ShapeDtypeStruct((NW, D), jnp.float32),
    scratch_types=[
        pltpu.VMEM((PER_TILE,), jnp.int32),
        pltpu.VMEM((CH, D), jnp.float32),
        pltpu.VMEM((CH, D), jnp.float32),
        pltpu.VMEM((CH, D), jnp.float32),
        pltpu.VMEM((CH, D), jnp.float32),
        pltpu.VMEM((1, D), jnp.float32),
        pltpu.SemaphoreType.DMA,
        pltpu.SemaphoreType.DMA,
        pltpu.SemaphoreType.DMA,
        pltpu.SemaphoreType.DMA,
    ],
)(_pool_body)


def _head_body(p_ref, mask_ref, row0_ref, w1_ref, b1_ref, w2_ref, b2_ref,
               out_ref):
  p = p_ref[...]
  r1 = p[0:16] + p[16:32]
  r2 = r1[0:8] + r1[8:16]
  feats = r2[0:4] + r2[4:8]

  msum = jnp.sum(mask_ref[...].astype(jnp.float32), axis=1, keepdims=True)
  feats = feats - (float(S) - msum) * row0_ref[...]
  feats = feats / jnp.clip(msum, 1e-6, None)

  h = jnp.dot(feats, w1_ref[...], preferred_element_type=jnp.float32)
  h = h + b1_ref[...]
  h = h * jax.nn.sigmoid(h)
  out_ref[...] = jnp.sum(h * w2_ref[...], axis=1, keepdims=True) + b2_ref[...]


_head = pl.pallas_call(
    _head_body,
    out_shape=jax.ShapeDtypeStruct((B, 1), jnp.float32),
)


def kernel(input_ids, attention_mask, table, W1, b1, W2, b2):
  mask = attention_mask.astype(jnp.int32)
  ids = jnp.where(mask != 0, input_ids.astype(jnp.int32), 0).reshape(-1)
  partials = _pool(ids, table)
  out = _head(
      partials,
      mask,
      table[0:1, :],
      W1,
      b1.reshape(1, H),
      W2.reshape(1, H),
      b2.reshape(1, 1),
  )
  return out.reshape(B)


# speedup vs baseline: 1.0354x; 1.0199x over previous
import functools

import jax
import jax.numpy as jnp
from jax import lax
from jax.experimental import pallas as pl
from jax.experimental.pallas import tpu as pltpu
from jax.experimental.pallas import tpu_sc as plsc

B, S, D, H = 4, 8192, 1024, 1024
NW = 32
PER_TILE = (B * S) // NW
CH = 16
NCH = PER_TILE // CH
DCH = D // 16
NACC = 8


def _pool_body(ids_hbm, msk_hbm, table_hbm, out_hbm, idx_v, msk_v, rows0_v,
               rows1_v, rows2_v, rows3_v, acc_v, sem0, sem1, sem2, sem3,
               isem):
  c = lax.axis_index("c")
  s = lax.axis_index("s")
  w = s * 2 + c
  b = w % B
  k = w // B
  base = b * S + k * PER_TILE

  bufs = (rows0_v, rows1_v, rows2_v, rows3_v)
  sems = (sem0, sem1, sem2, sem3)

  pltpu.async_copy(ids_hbm.at[pl.ds(base, PER_TILE)], idx_v, isem)
  pltpu.async_copy(msk_hbm.at[pl.ds(base, PER_TILE)], msk_v, isem)

  @plsc.parallel_loop(0, DCH, 1, unroll=4)
  def zero_body(ci):
    acc_v[0, pl.ds(ci * 16, 16)] = jnp.zeros((16,), jnp.float32)

  pltpu.make_async_copy(ids_hbm.at[pl.ds(base, PER_TILE)], idx_v, isem).wait()
  pltpu.make_async_copy(msk_hbm.at[pl.ds(base, PER_TILE)], msk_v, isem).wait()

  @plsc.parallel_loop(0, PER_TILE // 16, 1, unroll=4)
  def sel_body(i):
    sl = pl.ds(i * 16, 16)
    idx_v[sl] = jnp.where(msk_v[sl] != 0, idx_v[sl], 0)

  def start(j, buf, sem):
    pltpu.async_copy(table_hbm.at[idx_v.at[pl.ds(j * CH, CH)]], buf, sem)

  def wait(j, buf, sem):
    pltpu.make_async_copy(
        table_hbm.at[idx_v.at[pl.ds(j * CH, CH)]], buf, sem).wait()

  def accumulate(buf):
    @plsc.parallel_loop(0, DCH, 1, unroll=2)
    def col_body(ci):
      sl = pl.ds(ci * 16, 16)
      vs = [buf[r, sl] for r in range(NACC)]
      for r in range(NACC, CH):
        vs[r % NACC] = vs[r % NACC] + buf[r, sl]
      t0 = (vs[0] + vs[1]) + (vs[2] + vs[3])
      t1 = (vs[4] + vs[5]) + (vs[6] + vs[7])
      plsc.addupdate(acc_v.at[0, sl], t0 + t1)

  for p in range(4):
    start(p, bufs[p], sems[p])

  def ring_body(t, carry):
    j = 4 * t
    for q in range(4):
      jq = j + q
      wait(jq, bufs[q], sems[q])
      accumulate(bufs[q])

      @pl.when(jq + 4 < NCH)
      def _():
        start(jq + 4, bufs[q], sems[q])

    return carry

  lax.fori_loop(0, NCH // 4, ring_body, 0)
  pltpu.sync_copy(acc_v, out_hbm.at[pl.ds(w, 1)])


_pool = functools.partial(
    pl.kernel,
    mesh=plsc.VectorSubcoreMesh(core_axis_name="c", subcore_axis_name="s"),
    out_type=jax.ShapeDtypeStruct((NW, D), jnp.float32),
    scratch_types=(
        [pltpu.VMEM((PER_TILE,), jnp.int32),
         pltpu.VMEM((PER_TILE,), jnp.int32)]
        + [pltpu.VMEM((CH, D), jnp.float32) for _ in range(4)]
        + [pltpu.VMEM((1, D), jnp.float32)]
        + [pltpu.SemaphoreType.DMA for _ in range(5)]
    ),
)(_pool_body)


def _head_body(p_ref, mask_ref, row0_ref, w1_ref, b1_ref, w2_ref, b2_ref,
               out_ref):
  p = p_ref[...]
  r1 = p[0:16] + p[16:32]
  r2 = r1[0:8] + r1[8:16]
  feats = r2[0:4] + r2[4:8]

  msum = jnp.sum(mask_ref[...].astype(jnp.float32), axis=1, keepdims=True)
  feats = feats - (float(S) - msum) * row0_ref[...]
  feats = feats / jnp.clip(msum, 1e-6, None)

  h = jnp.dot(feats, w1_ref[...], preferred_element_type=jnp.float32)
  h = h + b1_ref[...]
  h = h * jax.nn.sigmoid(h)
  out_ref[...] = jnp.sum(h * w2_ref[...], axis=1, keepdims=True) + b2_ref[...]


_head = pl.pallas_call(
    _head_body,
    out_shape=jax.ShapeDtypeStruct((B, 1), jnp.float32),
)


def kernel(input_ids, attention_mask, table, W1, b1, W2, b2):
  mask = attention_mask.astype(jnp.int32)
  ids = input_ids.astype(jnp.int32).reshape(-1)
  partials = _pool(ids, mask.reshape(-1), table)
  out = _head(
      partials,
      mask,
      table[0:1, :],
      W1,
      b1.reshape(1, H),
      W2.reshape(1, H),
      b2.reshape(1, 1),
  )
  return out.reshape(B)
